# Initial kernel scaffold; baseline (speedup 1.0000x reference)
import jax
import jax.numpy as jnp
from jax.experimental import pallas as pl


def kernel(x, edge_index, W, b):
    raise NotImplementedError("write your pallas kernel here")


# speedup vs baseline: 18.1719x; 18.1719x over previous
import functools

import jax
import jax.numpy as jnp
from jax import lax
from jax.experimental import pallas as pl
from jax.experimental.pallas import tpu as pltpu
from jax.experimental.pallas import tpu_sc as plsc

NC = 2
NS = 16
NW = NC * NS

CH = 80


def _mesh():
    return plsc.VectorSubcoreMesh(
        core_axis_name="c", subcore_axis_name="s", num_cores=NC, num_subcores=NS
    )


def _fill_f32(ref, n, value):
    v = jnp.full((16,), value, dtype=jnp.float32)

    def body(i, _):
        ref[pl.ds(i * 16, 16)] = v
        return 0

    lax.fori_loop(0, n // 16, body, 0)


def _deg_call(dst, n_pad, e):
    ew = e // NW
    nch = ew // CH
    npt = n_pad // NS

    @functools.partial(
        pl.kernel,
        mesh=_mesh(),
        out_type=jax.ShapeDtypeStruct((NC * n_pad,), jnp.float32),
        scratch_types=[
            pltpu.VMEM((CH,), jnp.int32),
            pltpu.VMEM((CH,), jnp.float32),
            pltpu.VMEM((npt,), jnp.float32),
            pltpu.VMEM_SHARED((n_pad,), jnp.float32),
        ],
    )
    def deg_kernel(dst_hbm, deg_out, idx_v, ones_v, zrow_v, deg_sh):
        cid = lax.axis_index("c")
        sid = lax.axis_index("s")
        wid = cid * NS + sid

        _fill_f32(ones_v, CH, 1.0)
        _fill_f32(zrow_v, npt, 0.0)
        pltpu.sync_copy(zrow_v, deg_sh.at[pl.ds(sid * npt, npt)])
        plsc.subcore_barrier()

        base = wid * ew

        def body(j, _):
            off = pl.multiple_of(base + j * CH, 8)
            pltpu.sync_copy(dst_hbm.at[pl.ds(off, CH)], idx_v)
            pltpu.sync_copy(ones_v, deg_sh.at[idx_v], add=True)
            return 0

        lax.fori_loop(0, nch, body, 0)
        plsc.subcore_barrier()
        out_off = pl.multiple_of(cid * n_pad + sid * npt, 128)
        pltpu.sync_copy(
            deg_sh.at[pl.ds(sid * npt, npt)],
            deg_out.at[pl.ds(out_off, npt)],
        )

    return deg_kernel(dst)


def _agg_call(src, dst, g, n_pad, d, e):
    ew = e // NW
    nch = ew // CH
    npt = n_pad // NS
    nz = 128

    @functools.partial(
        pl.kernel,
        mesh=_mesh(),
        out_type=jax.ShapeDtypeStruct((NC, n_pad, d), jnp.float32),
        scratch_types=[
            pltpu.VMEM((CH,), jnp.int32),
            pltpu.VMEM((CH,), jnp.int32),
            pltpu.VMEM((CH, d), jnp.float32),
            pltpu.VMEM((nz, d), jnp.float32),
            pltpu.VMEM_SHARED((n_pad, d), jnp.float32),
            pltpu.SemaphoreType.DMA,
        ],
    )
    def agg_kernel(src_hbm, dst_hbm, g_hbm, acc_out,
                   idx_s, idx_d, rows_v, zbuf_v, acc_sh, sem):
        cid = lax.axis_index("c")
        sid = lax.axis_index("s")
        wid = cid * NS + sid

        zv = jnp.zeros((16,), dtype=jnp.float32)

        def zrow(i, _):
            def zcol(j, _):
                zbuf_v[i, pl.ds(j * 16, 16)] = zv
                return 0
            lax.fori_loop(0, d // 16, zcol, 0)
            return 0

        lax.fori_loop(0, nz, zrow, 0)

        def zslab(t, _):
            pltpu.sync_copy(zbuf_v, acc_sh.at[pl.ds(sid * npt + t * nz, nz)])
            return 0

        lax.fori_loop(0, npt // nz, zslab, 0)
        plsc.subcore_barrier()

        base = wid * ew

        def body(j, _):
            off = pl.multiple_of(base + j * CH, 8)
            pltpu.sync_copy(src_hbm.at[pl.ds(off, CH)], idx_s)
            pltpu.sync_copy(dst_hbm.at[pl.ds(off, CH)], idx_d)
            pltpu.async_copy(g_hbm.at[idx_s], rows_v, sem).wait()
            pltpu.sync_copy(rows_v, acc_sh.at[idx_d], add=True)
            return 0

        lax.fori_loop(0, nch, body, 0)
        plsc.subcore_barrier()
        pltpu.sync_copy(
            acc_sh.at[pl.ds(sid * npt, npt)],
            acc_out.at[cid, pl.ds(sid * npt, npt)],
        )

    return agg_kernel(src, dst, g)


def _prep_kernel(x_ref, w_ref, degt_ref, b_ref, g_ref, base_ref, dis_ref):
    deg = degt_ref[:, 0:1] + degt_ref[:, 1:2] + 1.0
    h = jnp.dot(x_ref[...], w_ref[...], preferred_element_type=jnp.float32)
    dis = lax.rsqrt(deg)
    g_ref[...] = h * dis
    base_ref[...] = h / deg + b_ref[...]
    dis_ref[...] = dis


def _final_kernel(a0_ref, a1_ref, dis_ref, base_ref, out_ref):
    out_ref[...] = (a0_ref[...] + a1_ref[...]) * dis_ref[...] + base_ref[...]


def kernel(x, edge_index, W, b):
    n, d_in = x.shape
    d_out = W.shape[1]
    e = edge_index.shape[1]
    src = edge_index[0]
    dst = edge_index[1]

    align = NS * 128
    n_pad = ((n + align - 1) // align) * align
    deg_flat = _deg_call(dst, n_pad, e)
    degt = jnp.transpose(deg_flat.reshape(NC, n_pad))

    rows = 2000
    grid = (n // rows,)
    g, base, dis = pl.pallas_call(
        _prep_kernel,
        grid=grid,
        in_specs=[
            pl.BlockSpec((rows, d_in), lambda i: (i, 0)),
            pl.BlockSpec((d_in, d_out), lambda i: (0, 0)),
            pl.BlockSpec((rows, 2), lambda i: (i, 0)),
            pl.BlockSpec((1, d_out), lambda i: (0, 0)),
        ],
        out_specs=[
            pl.BlockSpec((rows, d_out), lambda i: (i, 0)),
            pl.BlockSpec((rows, d_out), lambda i: (i, 0)),
            pl.BlockSpec((rows, 1), lambda i: (i, 0)),
        ],
        out_shape=[
            jax.ShapeDtypeStruct((n, d_out), jnp.float32),
            jax.ShapeDtypeStruct((n, d_out), jnp.float32),
            jax.ShapeDtypeStruct((n, 1), jnp.float32),
        ],
    )(x, W, degt, b.reshape(1, d_out))

    acc_parts = _agg_call(src, dst, g, n_pad, d_out, e)

    out = pl.pallas_call(
        _final_kernel,
        grid=grid,
        in_specs=[
            pl.BlockSpec((rows, d_out), lambda i: (i, 0)),
            pl.BlockSpec((rows, d_out), lambda i: (i, 0)),
            pl.BlockSpec((rows, 1), lambda i: (i, 0)),
            pl.BlockSpec((rows, d_out), lambda i: (i, 0)),
        ],
        out_specs=pl.BlockSpec((rows, d_out), lambda i: (i, 0)),
        out_shape=jax.ShapeDtypeStruct((n, d_out), jnp.float32),
    )(acc_parts[0], acc_parts[1], dis, base)

    return out


# speedup vs baseline: 23.4788x; 1.2920x over previous
import functools

import jax
import jax.numpy as jnp
from jax import lax
from jax.experimental import pallas as pl
from jax.experimental.pallas import tpu as pltpu
from jax.experimental.pallas import tpu_sc as plsc

NC = 2
NS = 16
NW = NC * NS

CH = 80


def _mesh():
    return plsc.VectorSubcoreMesh(
        core_axis_name="c", subcore_axis_name="s", num_cores=NC, num_subcores=NS
    )


def _fill_f32(ref, n, value):
    v = jnp.full((16,), value, dtype=jnp.float32)

    def body(i, _):
        ref[pl.ds(i * 16, 16)] = v
        return 0

    lax.fori_loop(0, n // 16, body, 0)


def _deg_call(dst_flat, n_pad, nch):
    npt = n_pad // NS

    @functools.partial(
        pl.kernel,
        mesh=_mesh(),
        out_type=jax.ShapeDtypeStruct((NC * n_pad,), jnp.float32),
        scratch_types=[
            pltpu.VMEM((CH,), jnp.int32),
            pltpu.VMEM((CH,), jnp.int32),
            pltpu.VMEM((CH,), jnp.float32),
            pltpu.VMEM((npt,), jnp.float32),
            pltpu.VMEM_SHARED((n_pad,), jnp.float32),
            pltpu.SemaphoreType.DMA,
        ],
    )
    def deg_kernel(dst_hbm, deg_out, idx_a, idx_b, ones_v, zrow_v, deg_sh,
                   sem):
        cid = lax.axis_index("c")
        sid = lax.axis_index("s")
        wid = cid * NS + sid

        _fill_f32(ones_v, CH, 1.0)
        _fill_f32(zrow_v, npt, 0.0)
        pltpu.sync_copy(zrow_v, deg_sh.at[pl.ds(sid * npt, npt)])
        plsc.subcore_barrier()

        ew = nch * CH
        base = wid * ew

        pltpu.sync_copy(dst_hbm.at[pl.ds(pl.multiple_of(base, 8), CH)], idx_a)

        def body(m, _):
            c = 2 * m
            db = pltpu.async_copy(
                dst_hbm.at[pl.ds(pl.multiple_of(base + (c + 1) * CH, 8), CH)],
                idx_b, sem)
            pltpu.sync_copy(ones_v, deg_sh.at[idx_a], add=True)
            db.wait()
            da = pltpu.async_copy(
                dst_hbm.at[pl.ds(pl.multiple_of(base + (c + 2) * CH, 8), CH)],
                idx_a, sem)
            pltpu.sync_copy(ones_v, deg_sh.at[idx_b], add=True)
            da.wait()
            return 0

        lax.fori_loop(0, (nch - 1) // 2, body, 0)
        pltpu.sync_copy(ones_v, deg_sh.at[idx_a], add=True)

        plsc.subcore_barrier()
        out_off = pl.multiple_of(cid * n_pad + sid * npt, 128)
        pltpu.sync_copy(
            deg_sh.at[pl.ds(sid * npt, npt)],
            deg_out.at[pl.ds(out_off, npt)],
        )

    return deg_kernel(dst_flat)


def _agg_call(epk, g, n_pad, d, nch):
    npt = n_pad // NS

    @functools.partial(
        pl.kernel,
        mesh=_mesh(),
        out_type=jax.ShapeDtypeStruct((NC, n_pad, d), jnp.float32),
        scratch_types=[
            pltpu.VMEM((CH,), jnp.int32),
            pltpu.VMEM((CH,), jnp.int32),
            pltpu.VMEM((CH,), jnp.int32),
            pltpu.VMEM((CH,), jnp.int32),
            pltpu.VMEM((CH,), jnp.int32),
            pltpu.VMEM((CH, d), jnp.float32),
            pltpu.VMEM((CH, d), jnp.float32),
            pltpu.VMEM_SHARED((n_pad, d), jnp.float32),
            pltpu.SemaphoreType.DMA,
            pltpu.SemaphoreType.DMA,
        ],
    )
    def agg_kernel(epk_hbm, g_hbm, acc_out,
                   pidx, sidx_a, didx_a, sidx_b, didx_b,
                   rows_a, rows_b, acc_sh, sem_a, sem_b):
        cid = lax.axis_index("c")
        sid = lax.axis_index("s")
        wid = cid * NS + sid
        ew = nch * CH
        base = wid * ew

        def load_unpack(c, s_ref, d_ref):
            off = pl.multiple_of(base + c * CH, 8)
            pltpu.sync_copy(epk_hbm.at[pl.ds(off, CH)], pidx)
            for i in range(CH // 16):
                v = pidx[pl.ds(i * 16, 16)]
                s_ref[pl.ds(i * 16, 16)] = jnp.right_shift(v, 16)
                d_ref[pl.ds(i * 16, 16)] = jnp.bitwise_and(v, 0xFFFF)

        zv = jnp.zeros((16,), dtype=jnp.float32)

        def zrow(i, _):
            def zcol(jj, _):
                rows_a[i, pl.ds(jj * 16, 16)] = zv
                return 0
            lax.fori_loop(0, d // 16, zcol, 0)
            return 0

        lax.fori_loop(0, CH, zrow, 0)

        def zslab(t, _):
            pltpu.sync_copy(
                rows_a, acc_sh.at[pl.ds(sid * npt + t * CH, CH)])
            return 0

        lax.fori_loop(0, npt // CH, zslab, 0)
        plsc.subcore_barrier()

        load_unpack(0, sidx_a, didx_a)
        pltpu.async_copy(g_hbm.at[sidx_a], rows_a, sem_a).wait()

        def body(m, _):
            c = 2 * m
            load_unpack(c + 1, sidx_b, didx_b)
            db = pltpu.async_copy(g_hbm.at[sidx_b], rows_b, sem_b)
            pltpu.sync_copy(rows_a, acc_sh.at[didx_a], add=True)
            db.wait()
            load_unpack(c + 2, sidx_a, didx_a)
            da = pltpu.async_copy(g_hbm.at[sidx_a], rows_a, sem_a)
            pltpu.sync_copy(rows_b, acc_sh.at[didx_b], add=True)
            da.wait()
            return 0

        lax.fori_loop(0, (nch - 1) // 2, body, 0)
        pltpu.sync_copy(rows_a, acc_sh.at[didx_a], add=True)

        plsc.subcore_barrier()
        pltpu.sync_copy(
            acc_sh.at[pl.ds(sid * npt, npt)],
            acc_out.at[cid, pl.ds(sid * npt, npt)],
        )

    return agg_kernel(epk, g)


def _prep_kernel(x_ref, w_ref, degt_ref, b_ref, ei_ref,
                 g_ref, base_ref, dis_ref, epk_ref):
    deg = degt_ref[:, 0:1] + degt_ref[:, 1:2] + 1.0
    h = jnp.dot(x_ref[...], w_ref[...], preferred_element_type=jnp.float32)
    dis = lax.rsqrt(deg)
    g_ref[...] = h * dis
    base_ref[...] = h / deg + b_ref[...]
    dis_ref[...] = dis
    epk_ref[...] = jnp.left_shift(ei_ref[0:1, :], 16) | ei_ref[1:2, :]


def _final_kernel(a0_ref, a1_ref, dis_ref, base_ref, out_ref):
    out_ref[...] = (a0_ref[...] + a1_ref[...]) * dis_ref[...] + base_ref[...]


def kernel(x, edge_index, W, b):
    n, d_in = x.shape
    d_out = W.shape[1]
    e = edge_index.shape[1]
    ew = e // NW
    nch = ew // CH
    dst = edge_index[1]

    align = NS * 128
    n_pad = ((n + align - 1) // align) * align
    deg_flat = _deg_call(dst, n_pad, nch)
    degt = jnp.transpose(deg_flat.reshape(NC, n_pad))

    rows = 2000
    eb = e // (n // rows)
    grid = (n // rows,)
    g, base, dis, epk2 = pl.pallas_call(
        _prep_kernel,
        grid=grid,
        in_specs=[
            pl.BlockSpec((rows, d_in), lambda i: (i, 0)),
            pl.BlockSpec((d_in, d_out), lambda i: (0, 0)),
            pl.BlockSpec((rows, 2), lambda i: (i, 0)),
            pl.BlockSpec((1, d_out), lambda i: (0, 0)),
            pl.BlockSpec((2, eb), lambda i: (0, i)),
        ],
        out_specs=[
            pl.BlockSpec((rows, d_out), lambda i: (i, 0)),
            pl.BlockSpec((rows, d_out), lambda i: (i, 0)),
            pl.BlockSpec((rows, 1), lambda i: (i, 0)),
            pl.BlockSpec((1, eb), lambda i: (0, i)),
        ],
        out_shape=[
            jax.ShapeDtypeStruct((n, d_out), jnp.float32),
            jax.ShapeDtypeStruct((n, d_out), jnp.float32),
            jax.ShapeDtypeStruct((n, 1), jnp.float32),
            jax.ShapeDtypeStruct((1, e), jnp.int32),
        ],
    )(x, W, degt, b.reshape(1, d_out), edge_index)

    acc_parts = _agg_call(epk2.reshape(e), g, n_pad, d_out, nch)

    out = pl.pallas_call(
        _final_kernel,
        grid=grid,
        in_specs=[
            pl.BlockSpec((rows, d_out), lambda i: (i, 0)),
            pl.BlockSpec((rows, d_out), lambda i: (i, 0)),
            pl.BlockSpec((rows, 1), lambda i: (i, 0)),
            pl.BlockSpec((rows, d_out), lambda i: (i, 0)),
        ],
        out_specs=pl.BlockSpec((rows, d_out), lambda i: (i, 0)),
        out_shape=jax.ShapeDtypeStruct((n, d_out), jnp.float32),
    )(acc_parts[0], acc_parts[1], dis, base)

    return out
